# baseline (speedup 1.0000x reference)
import functools
import math

import jax
import jax.numpy as jnp
from jax import lax
from jax.experimental import pallas as pl
from jax.experimental.pallas import tpu as pltpu
from jax.experimental.pallas import tpu_sc as plsc

_B, _C, _T, _H, _W = 16, 768, 8, 12, 30
_N = _H * _W
_K = 120
_R = 2 + _K + _N
_HALF = _R // 2
_NW = 32
_CHUNK = 64


def _scores_kernel(x_hbm, cls_ref, p_ref, s0, sem):
    i = pl.program_id(0)
    cp = pltpu.make_async_copy(x_hbm.at[i, :, 0, :], s0, sem)
    cp.start()
    cp.wait()
    x0t = s0[...]
    cls0 = cls_ref[0, 0:1, :]
    s = jax.lax.dot_general(
        cls0, x0t, (((1,), (1,)), ((), ())),
        preferred_element_type=jnp.float32) / math.sqrt(_C)
    p_ref[0] = jax.nn.softmax(s, axis=-1)


def _topk_kernel(p_all_ref, idx_ref):
    norm = jnp.max(p_all_ref[...])
    col = jax.lax.broadcasted_iota(jnp.int32, (_N, _N), 1)
    row = jax.lax.broadcasted_iota(jnp.int32, (_N, _N), 0)
    k_iota = jax.lax.broadcasted_iota(jnp.int32, (_K, _N), 0)
    n_iota = jax.lax.broadcasted_iota(jnp.int32, (_K, _N), 1)
    for b in range(_B):
        q = p_all_ref[b] / norm
        qT = jnp.transpose(q)
        cmp = (qT > q) | ((qT == q) & (row < col))
        rank = jnp.sum(cmp.astype(jnp.int32), axis=0, keepdims=True)
        onehot = (k_iota == rank).astype(jnp.int32)
        ids = jnp.sum(onehot * n_iota, axis=1, keepdims=True)
        idx_ref[b, 0:_K, :] = ids
        idx_ref[b, _K:, :] = jnp.zeros((128 - _K, 1), jnp.int32)


def _make_assemble():
    mesh = plsc.VectorSubcoreMesh(core_axis_name="c", subcore_axis_name="s")

    @functools.partial(
        pl.kernel,
        mesh=mesh,
        out_type=jax.ShapeDtypeStruct((_B * _R, _C), jnp.float32),
        scratch_types=[
            pltpu.VMEM((4, _CHUNK), jnp.int32),
            pltpu.VMEM((4, _CHUNK), jnp.int32),
            pltpu.VMEM((_CHUNK, _C), jnp.float32),
            pltpu.SemaphoreType.DMA,
            pltpu.SemaphoreType.DMA,
        ],
    )
    def _assemble(xflat_hbm, cls_hbm, src_hbm, dst_hbm, out_hbm,
                  src_v, dst_v, rows_v, gsem, ssem):
        cid = lax.axis_index("c")
        sid = lax.axis_index("s")
        w = sid * 2 + cid
        pltpu.sync_copy(src_hbm.at[w], src_v)
        pltpu.sync_copy(dst_hbm.at[w], dst_v)
        for j in range(4):
            pltpu.async_copy(xflat_hbm.at[src_v.at[j]], rows_v, gsem).wait()
            if j == 0:
                @pl.when(cid == 0)
                def _():
                    pltpu.sync_copy(cls_hbm.at[sid * 8], rows_v.at[0])
            if j == 1:
                @pl.when(cid == 0)
                def _():
                    pltpu.sync_copy(cls_hbm.at[sid * 8 + 1], rows_v.at[57])
            pltpu.async_copy(rows_v, out_hbm.at[dst_v.at[j]], ssem).wait()

    return _assemble


def kernel(x, cls_tokens):
    b, c, t, h, w = x.shape
    n = h * w
    xt4 = jnp.transpose(x, (0, 3, 4, 2, 1)).reshape(b, n, t, c)
    xflat = xt4.reshape(b * n * t, c)
    cls_flat = cls_tokens.reshape(b * t, c)

    p = pl.pallas_call(
        _scores_kernel,
        grid=(b,),
        in_specs=[
            pl.BlockSpec(memory_space=pl.ANY),
            pl.BlockSpec((1, t, c), lambda i: (i, 0, 0)),
        ],
        out_specs=pl.BlockSpec((1, 1, n), lambda i: (i, 0, 0)),
        out_shape=jax.ShapeDtypeStruct((b, 1, n), jnp.float32),
        scratch_shapes=[
            pltpu.VMEM((n, c), jnp.float32),
            pltpu.SemaphoreType.DMA,
        ],
    )(xt4, cls_tokens)

    sel = pl.pallas_call(
        _topk_kernel,
        in_specs=[pl.BlockSpec((b, 1, n), lambda: (0, 0, 0))],
        out_specs=pl.BlockSpec((b, 128, 1), lambda: (0, 0, 0)),
        out_shape=jax.ShapeDtypeStruct((b, 128, 1), jnp.int32),
    )(p)
    sel_ids = sel[:, :_K, 0]

    batch_base = (jnp.arange(b, dtype=jnp.int32) * (n * t))[:, None]
    sel_rows = batch_base + sel_ids * t
    glob_rows = batch_base + jnp.arange(n, dtype=jnp.int32)[None, :] * t + 1
    zero = jnp.zeros((b, 1), jnp.int32)
    row_map = jnp.concatenate(
        [batch_base + zero, sel_rows, batch_base + zero, glob_rows], axis=1)
    row_map = row_map.reshape(_NW, _HALF)

    j_idx = jnp.minimum(
        jnp.arange(4, dtype=jnp.int32)[:, None] * _CHUNK
        + jnp.arange(_CHUNK, dtype=jnp.int32)[None, :],
        _HALF - 1)
    src_map = jnp.take_along_axis(
        row_map[:, None, :], j_idx[None], axis=2)
    dst_map = (jnp.arange(_NW, dtype=jnp.int32) * _HALF)[:, None, None] \
        + j_idx[None]

    out_flat = _make_assemble()(xflat, cls_flat, src_map, dst_map)
    return out_flat.reshape(b, _R, c)


# speedup vs baseline: 1.2194x; 1.2194x over previous
import functools
import math

import jax
import jax.numpy as jnp
from jax import lax
from jax.experimental import pallas as pl
from jax.experimental.pallas import tpu as pltpu
from jax.experimental.pallas import tpu_sc as plsc

_B, _C, _T, _H, _W = 16, 768, 8, 12, 30
_N = _H * _W
_K = 120
_R = 2 + _K + _N
_HALF = _R // 2
_NW = 32
_CHUNK = 64


def _scores_kernel(x_hbm, cls_ref, p_ref, s0, sem):
    i = pl.program_id(0)
    cp = pltpu.make_async_copy(x_hbm.at[i, :, 0, :], s0, sem)
    cp.start()
    cp.wait()
    x0t = s0[...]
    cls0 = cls_ref[0, 0:1, :]
    s = jax.lax.dot_general(
        cls0, x0t, (((1,), (1,)), ((), ())),
        preferred_element_type=jnp.float32) / math.sqrt(_C)
    p_ref[0] = jax.nn.softmax(s, axis=-1)


def _topk_kernel(p_all_ref, idx_ref):
    norm = jnp.max(p_all_ref[...])
    col = jax.lax.broadcasted_iota(jnp.int32, (_N, _N), 1)
    row = jax.lax.broadcasted_iota(jnp.int32, (_N, _N), 0)
    k_iota = jax.lax.broadcasted_iota(jnp.int32, (_K, _N), 0)
    n_iota = jax.lax.broadcasted_iota(jnp.int32, (_K, _N), 1)
    for b in range(_B):
        q = p_all_ref[b] / norm
        qT = jnp.transpose(q)
        cmp = (qT > q) | ((qT == q) & (row < col))
        rank = jnp.sum(cmp.astype(jnp.int32), axis=0, keepdims=True)
        onehot = (k_iota == rank).astype(jnp.int32)
        ids = jnp.sum(onehot * n_iota, axis=1, keepdims=True)
        idx_ref[b, 0:_K, :] = ids
        idx_ref[b, _K:, :] = jnp.zeros((128 - _K, 1), jnp.int32)


def _make_assemble():
    mesh = plsc.VectorSubcoreMesh(core_axis_name="c", subcore_axis_name="s")

    @functools.partial(
        pl.kernel,
        mesh=mesh,
        out_type=jax.ShapeDtypeStruct((_B, _R, _C), jnp.float32),
        scratch_types=[
            pltpu.VMEM((4, _CHUNK), jnp.int32),
            pltpu.VMEM((4, _CHUNK), jnp.int32),
            pltpu.VMEM((_CHUNK, _C), jnp.float32),
            pltpu.SemaphoreType.DMA,
            pltpu.SemaphoreType.DMA,
        ],
    )
    def _assemble(xflat_hbm, cls_hbm, src_hbm, dst_hbm, out_hbm,
                  src_v, dst_v, rows_v, gsem, ssem):
        cid = lax.axis_index("c")
        sid = lax.axis_index("s")
        w = sid * 2 + cid
        pltpu.sync_copy(src_hbm.at[w], src_v)
        pltpu.sync_copy(dst_hbm.at[w], dst_v)
        for j in range(4):
            pltpu.async_copy(xflat_hbm.at[src_v.at[j]], rows_v, gsem).wait()
            if j == 0:
                @pl.when(cid == 0)
                def _():
                    pltpu.sync_copy(cls_hbm.at[sid * 8], rows_v.at[0])
            if j == 1:
                @pl.when(cid == 0)
                def _():
                    pltpu.sync_copy(cls_hbm.at[sid * 8 + 1], rows_v.at[57])
            pltpu.async_copy(rows_v, out_hbm.at[sid].at[dst_v.at[j]],
                             ssem).wait()

    return _assemble


def kernel(x, cls_tokens):
    b, c, t, h, w = x.shape
    n = h * w
    xt4 = jnp.transpose(x, (0, 3, 4, 2, 1)).reshape(b, n, t, c)
    xflat = xt4.reshape(b * n * t, c)
    cls_flat = cls_tokens.reshape(b * t, c)

    p = pl.pallas_call(
        _scores_kernel,
        grid=(b,),
        in_specs=[
            pl.BlockSpec(memory_space=pl.ANY),
            pl.BlockSpec((1, t, c), lambda i: (i, 0, 0)),
        ],
        out_specs=pl.BlockSpec((1, 1, n), lambda i: (i, 0, 0)),
        out_shape=jax.ShapeDtypeStruct((b, 1, n), jnp.float32),
        scratch_shapes=[
            pltpu.VMEM((n, c), jnp.float32),
            pltpu.SemaphoreType.DMA,
        ],
    )(xt4, cls_tokens)

    sel = pl.pallas_call(
        _topk_kernel,
        in_specs=[pl.BlockSpec((b, 1, n), lambda: (0, 0, 0))],
        out_specs=pl.BlockSpec((b, 128, 1), lambda: (0, 0, 0)),
        out_shape=jax.ShapeDtypeStruct((b, 128, 1), jnp.int32),
    )(p)
    sel_ids = sel[:, :_K, 0]

    batch_base = (jnp.arange(b, dtype=jnp.int32) * (n * t))[:, None]
    sel_rows = batch_base + sel_ids * t
    glob_rows = batch_base + jnp.arange(n, dtype=jnp.int32)[None, :] * t + 1
    zero = jnp.zeros((b, 1), jnp.int32)
    row_map = jnp.concatenate(
        [batch_base + zero, sel_rows, batch_base + zero, glob_rows], axis=1)
    row_map = row_map.reshape(_NW, _HALF)

    j_idx = jnp.minimum(
        jnp.arange(4, dtype=jnp.int32)[:, None] * _CHUNK
        + jnp.arange(_CHUNK, dtype=jnp.int32)[None, :],
        _HALF - 1)
    src_map = jnp.take_along_axis(
        row_map[:, None, :], j_idx[None], axis=2)
    dst_map = ((jnp.arange(_NW, dtype=jnp.int32) % 2) * _HALF)[:, None, None] \
        + j_idx[None]

    return _make_assemble()(xflat, cls_flat, src_map, dst_map)
